# baseline (speedup 1.0000x reference)
import functools

import jax
import jax.numpy as jnp
from jax import lax
from jax.experimental import pallas as pl
from jax.experimental.pallas import tpu as pltpu
from jax.experimental.pallas import tpu_sc as plsc

_NC = 2
_NS = 16
_LANES = 16


def _build(B, L, V, D):
    DPC = D // _NC
    BPT = B // _NS
    SEG = 65536
    assert _NS * SEG >= V
    NRUN = SEG // 128
    CL = 8
    NCHK = L // CL
    NBLK = (NCHK - 1) // 2

    mesh = plsc.VectorSubcoreMesh(core_axis_name="c", subcore_axis_name="s")

    scratch = [
        pltpu.VMEM_SHARED((_NS * SEG,), jnp.float32),
        pltpu.VMEM((L * BPT,), jnp.int32),
        pltpu.VMEM((CL * BPT,), jnp.float32),
        pltpu.VMEM((CL * BPT,), jnp.float32),
        pltpu.VMEM((CL, BPT), jnp.float32),
        pltpu.VMEM((256,), jnp.float32),
        pltpu.SemaphoreType.DMA,
        pltpu.SemaphoreType.DMA,
        pltpu.SemaphoreType.DMA,
        pltpu.SemaphoreType.DMA,
        pltpu.SemaphoreType.DMA,
    ]

    @functools.partial(
        pl.kernel,
        out_type=jax.ShapeDtypeStruct((L, D, B), jnp.float32),
        mesh=mesh,
        scratch_types=scratch,
    )
    def run(tok_hbm, tab_hbm, pos_hbm, out_hbm,
            row_sp, tok1d, gat0, gat1, wb, pos_v,
            ssem, tsem, gsem0, gsem1, osem):
        c = lax.axis_index("c")
        s = lax.axis_index("s")
        b0 = s * BPT
        d_base = c * DPC
        v0 = s * SEG
        gats = (gat0, gat1)
        gsems = (gsem0, gsem1)

        def fire_stage(dd):
            def one(j, c2):
                pltpu.async_copy(tab_hbm.at[dd, pl.ds(v0 + j * 128, 128)],
                                 row_sp.at[pl.ds(v0 + j * 128, 128)], ssem)
                return c2
            lax.fori_loop(0, NRUN, one, 0)

        def drain_stage(dd):
            def one(j, c2):
                pltpu.make_async_copy(
                    tab_hbm.at[dd, pl.ds(v0 + j * 128, 128)],
                    row_sp.at[pl.ds(v0 + j * 128, 128)], ssem).wait()
                return c2
            lax.fori_loop(0, NRUN, one, 0)

        fire_stage(d_base)

        def stage_tok(l, c2):
            pltpu.async_copy(tok_hbm.at[pl.ds(l * B + b0, BPT)],
                             tok1d.at[pl.ds(l * BPT, BPT)], tsem)
            return c2

        lax.fori_loop(0, L, stage_tok, 0)

        def drain_tok(l, c2):
            pltpu.make_async_copy(tok_hbm.at[pl.ds(l * B + b0, BPT)],
                                  tok1d.at[pl.ds(l * BPT, BPT)], tsem).wait()
            return c2

        lax.fori_loop(0, L, drain_tok, 0)

        def fire_gather(n, g):
            pltpu.async_copy(
                row_sp.at[tok1d.at[pl.ds(n * CL * BPT, CL * BPT)]],
                gats[g], gsems[g])

        def wait_gather(n, g):
            pltpu.make_async_copy(
                row_sp.at[tok1d.at[pl.ds(n * CL * BPT, CL * BPT)]],
                gats[g], gsems[g]).wait()

        def wb_push(n, dd):
            for li in range(CL):
                pltpu.async_copy(
                    wb.at[li], out_hbm.at[n * CL + li, dd, pl.ds(b0, BPT)],
                    osem)

        def wb_drain(n, dd):
            for li in range(CL):
                pltpu.make_async_copy(
                    wb.at[li], out_hbm.at[n * CL + li, dd, pl.ds(b0, BPT)],
                    osem).wait()

        def add_chunk(g, pv16, li0, first):
            @pl.when(jnp.logical_not(first))
            def _drain():
                wb_drain(0, 0)
            for li in range(CL):
                pv = jnp.full((_LANES,), pv16[li0 + li], dtype=jnp.float32)
                for j in range(BPT // _LANES):
                    sl = pl.ds((li * BPT) + j * _LANES, _LANES)
                    wb[li, pl.ds(j * _LANES, _LANES)] = gats[g][sl] + pv

        def phase(p, carry):
            dd = d_base + p
            drain_stage(dd)
            plsc.subcore_barrier()
            pltpu.sync_copy(pos_hbm.at[pl.ds(dd * L, L)],
                            pos_v.at[pl.ds(0, L)])
            fire_gather(0, 0)
            fire_gather(1, 1)

            def block(k, c2):
                pv16 = pos_v[pl.ds(k * _LANES, _LANES)]
                for i in range(2):
                    n = 2 * k + i
                    wait_gather(n, i)
                    add_chunk(i, pv16, i * CL, (p == 0) & (n == 0))
                    wb_push(n, dd)

                    @pl.when(n + 2 < NCHK)
                    def _ahead():
                        fire_gather(n + 2, i)
                return c2

            lax.fori_loop(0, NBLK, block, 0)
            wait_gather(NCHK - 1, 0)
            plsc.subcore_barrier()

            @pl.when(p + 1 < DPC)
            def _stage():
                fire_stage(dd + 1)

            pv16 = pos_v[pl.ds(L - _LANES, _LANES)]
            add_chunk(0, pv16, _LANES - CL, False)
            wb_push(NCHK - 1, dd)
            return carry

        lax.fori_loop(0, DPC, phase, 0)
        wb_drain(0, 0)

    return run


def kernel(tokens, token_table, pos_emb):
    B, L = tokens.shape
    V, D = token_table.shape
    run = _build(B, L, V, D)
    out_t = run(tokens.T.reshape(L * B), token_table.T,
                pos_emb.T.reshape(D * L))
    return jnp.transpose(out_t, (2, 0, 1))
